# baseline (speedup 1.0000x reference)
import functools

import jax
import jax.numpy as jnp
from jax import lax
from jax.experimental import pallas as pl
from jax.experimental.pallas import tpu as pltpu
from jax.experimental.pallas import tpu_sc as plsc

_LANES = 16
_CHUNK = 128
_WIDE = 128


_GDT = jnp.bfloat16
_EPW = 8


@functools.lru_cache(maxsize=None)
def _make_sc_gather(B, U, I):
    info = plsc.get_sparse_core_info()
    NC, NS = info.num_cores, info.num_subcores
    NW = NC * NS
    bpw = B // NW
    nch = bpw // _CHUNK
    assert bpw * NW == B and nch * _CHUNK == bpw

    mesh = plsc.VectorSubcoreMesh(core_axis_name="c", subcore_axis_name="s")

    @functools.partial(
        pl.kernel,
        mesh=mesh,
        out_type=(
            jax.ShapeDtypeStruct((B, _WIDE), jnp.int32),
            jax.ShapeDtypeStruct((B, _WIDE), jnp.int32),
        ),
        scratch_types=[
            pltpu.VMEM((nch, _CHUNK), jnp.int32),
            pltpu.VMEM((nch, _CHUNK), jnp.int32),
            pltpu.VMEM((bpw, _WIDE), jnp.int32),
            pltpu.SemaphoreType.DMA,
        ],
    )
    def sc_gather(uid_hbm, iid_hbm, uemb_hbm, iemb_hbm, ue_out, ie_out,
                  uidx, iidx, rows, sem):
        wid = lax.axis_index("s") * NC + lax.axis_index("c")
        row0 = wid * nch
        base = wid * bpw

        pltpu.sync_copy(uid_hbm.at[pl.ds(row0, nch)], uidx)
        pltpu.sync_copy(iid_hbm.at[pl.ds(row0, nch)], iidx)

        sh = _EPW.bit_length() - 1
        for r in range(nch):
            for j in range(_CHUNK // _LANES):
                sl = pl.ds(j * _LANES, _LANES)
                v = uidx[r, sl] - 1
                v = jnp.where(v < 0, v + U, v)
                uidx[r, sl] = lax.shift_right_logical(v, sh)
                w = iidx[r, sl] - 1
                w = jnp.where(w < 0, w + I, w)
                iidx[r, sl] = lax.shift_right_logical(w, sh)

        copies = []
        for r in range(nch):
            copies.append(pltpu.async_copy(
                uemb_hbm.at[uidx.at[r]], rows.at[pl.ds(r * _CHUNK, _CHUNK)],
                sem))
        for c in copies:
            c.wait()
        pltpu.sync_copy(rows, ue_out.at[pl.ds(base, bpw)])

        copies = []
        for r in range(nch):
            copies.append(pltpu.async_copy(
                iemb_hbm.at[iidx.at[r]], rows.at[pl.ds(r * _CHUNK, _CHUNK)],
                sem))
        for c in copies:
            c.wait()
        pltpu.sync_copy(rows, ie_out.at[pl.ds(base, bpw)])

    return sc_gather


def _part_select(wide, ids, table_rows):
    w = ids - 1
    w = jnp.where(w < 0, w + table_rows, w)
    q = jnp.bitwise_and(w, _EPW - 1)
    h = wide.shape[1] // _EPW
    out = None
    for qi in range(_EPW):
        m = (q == qi).astype(jnp.float32)
        part = m * wide[:, qi * h:(qi + 1) * h]
        out = part if out is None else out + part
    return out


def _mlp_body(U, I, uew_ref, iew_ref, xu_ref, xi_ref, tp_ref, g_ref, o_ref,
              gemb_ref, oemb_ref,
              w1u_ref, w1g_ref, w1o_ref, w1i_ref, w1t_ref, b1_ref,
              w2_ref, b2_ref, w3_ref, b3_ref, out_ref):
    f32 = jnp.float32
    dot = functools.partial(jnp.dot, preferred_element_type=f32)

    ue = _part_select(uew_ref[...].astype(f32), xu_ref[...], U)
    ie = _part_select(iew_ref[...].astype(f32), xi_ref[...], I)

    acc = dot(ue, w1u_ref[...])
    acc += dot(ie, w1i_ref[...])
    acc += dot(tp_ref[...], w1t_ref[...])

    g2 = dot(gemb_ref[...], w1g_ref[...])
    acc += jnp.where(g_ref[...] == 0, g2[0:1, :], g2[1:2, :])

    o2 = dot(oemb_ref[...], w1o_ref[...])
    blk = o_ref.shape[0]
    iota = lax.broadcasted_iota(jnp.int32, (blk, o2.shape[0]), 1)
    oh = (o_ref[...] == iota).astype(f32)
    acc += dot(oh, o2)

    h1 = jnp.maximum(acc + b1_ref[...], 0.0)
    h2 = jnp.maximum(dot(h1, w2_ref[...]) + b2_ref[...], 0.0)
    z = dot(h2, w3_ref[...]) + b3_ref[...]
    out_ref[...] = 1.0 / (1.0 + jnp.exp(-z))


def _mlp_call(B, blk, U, I, uew, iew, xu, xi, tp, g2d, o2d, gemb, oemb,
              w1u, w1g, w1o, w1i, w1t, b1, w2, b2, w3, b3):
    grid = (B // blk,)

    def row_spec(c):
        return pl.BlockSpec((blk, c), lambda i: (i, 0))

    def full_spec(shape):
        return pl.BlockSpec(shape, lambda i: (0,) * len(shape))

    return pl.pallas_call(
        functools.partial(_mlp_body, U, I),
        grid=grid,
        in_specs=[
            row_spec(uew.shape[1]), row_spec(iew.shape[1]),
            row_spec(1), row_spec(1), row_spec(tp.shape[1]),
            row_spec(1), row_spec(1),
            full_spec(gemb.shape), full_spec(oemb.shape),
            full_spec(w1u.shape), full_spec(w1g.shape), full_spec(w1o.shape),
            full_spec(w1i.shape), full_spec(w1t.shape), full_spec(b1.shape),
            full_spec(w2.shape), full_spec(b2.shape),
            full_spec(w3.shape), full_spec(b3.shape),
        ],
        out_specs=pl.BlockSpec((blk, 1), lambda i: (i, 0)),
        out_shape=jax.ShapeDtypeStruct((B, 1), jnp.float32),
    )(uew, iew, xu, xi, tp, g2d, o2d, gemb, oemb,
      w1u, w1g, w1o, w1i, w1t, b1, w2, b2, w3, b3)


def kernel(x, gender, occupation, type, user_emb, item_emb, gender_emb, occ_emb,
           W1, b1, W2, b2, W3, b3):
    B = x.shape[0]
    U, H = user_emb.shape
    I = item_emb.shape[0]

    xi32 = x.astype(jnp.int32)
    uid2d = xi32[:, 0].reshape(B // _CHUNK, _CHUNK)
    iid2d = xi32[:, 1].reshape(B // _CHUNK, _CHUNK)
    uemb_w = lax.bitcast_convert_type(
        user_emb.astype(_GDT).reshape(U, H // 2, 2),
        jnp.int32).reshape(U * H // (2 * _WIDE), _WIDE)
    iemb_w = lax.bitcast_convert_type(
        item_emb.astype(_GDT).reshape(I, H // 2, 2),
        jnp.int32).reshape(I * H // (2 * _WIDE), _WIDE)
    uew_i, iew_i = _make_sc_gather(B, U, I)(uid2d, iid2d, uemb_w, iemb_w)
    uew = lax.bitcast_convert_type(uew_i, _GDT).reshape(B, 2 * _WIDE)
    iew = lax.bitcast_convert_type(iew_i, _GDT).reshape(B, 2 * _WIDE)

    Hg = gender_emb.shape[1]
    Ho = occ_emb.shape[1]
    o0 = H + Hg
    i0 = o0 + Ho
    t0 = i0 + H
    return _mlp_call(
        B, 2048, U, I, uew, iew,
        xi32[:, 0].reshape(B, 1), xi32[:, 1].reshape(B, 1), type,
        gender.astype(jnp.int32).reshape(B, 1),
        occupation.astype(jnp.int32).reshape(B, 1),
        gender_emb, occ_emb,
        W1[:H], W1[H:o0], W1[o0:i0], W1[i0:t0], W1[t0:], b1.reshape(1, H),
        W2, b2.reshape(1, -1), W3, b3.reshape(1, 1))


# speedup vs baseline: 5.0902x; 5.0902x over previous
import functools

import jax
import jax.numpy as jnp
from jax import lax
from jax.experimental import pallas as pl
from jax.experimental.pallas import tpu as pltpu
from jax.experimental.pallas import tpu_sc as plsc

_LANES = 16
_CHUNK = 16


@functools.lru_cache(maxsize=None)
def _make_sc_gather(B, H, U, I):
    info = plsc.get_sparse_core_info()
    NC, NS = info.num_cores, info.num_subcores
    NW = NC * NS
    bpw = B // NW
    nch = bpw // _CHUNK
    ntr = H // 8
    assert bpw * NW == B and nch * _CHUNK == bpw

    mesh = plsc.VectorSubcoreMesh(core_axis_name="c", subcore_axis_name="s")

    @functools.partial(
        pl.kernel,
        mesh=mesh,
        compiler_params=pltpu.CompilerParams(needs_layout_passes=False),
        out_type=(
            jax.ShapeDtypeStruct((B, H), jnp.float32),
            jax.ShapeDtypeStruct((B, H), jnp.float32),
        ),
        scratch_types=[
            pltpu.VMEM((bpw,), jnp.int32),
            pltpu.VMEM((_CHUNK, 8, 128), jnp.float32),
            pltpu.VMEM((bpw, H), jnp.float32),
            pltpu.SemaphoreType.DMA,
            pltpu.SemaphoreType.DMA,
        ],
    )
    def sc_gather(uid_hbm, iid_hbm, uembT_hbm, iembT_hbm, ue_out, ie_out,
                  ids_v, stage, rows, sem, sem2):
        wid = lax.axis_index("s") * NC + lax.axis_index("c")
        base = pl.multiple_of(wid * bpw, bpw)

        def table(idx_hbm, tbl, out, nrows):
            pltpu.sync_copy(idx_hbm.at[pl.ds(base, bpw)], ids_v)
            for j in range(bpw // _LANES):
                sl = pl.ds(j * _LANES, _LANES)
                v = ids_v[sl] - 1
                ids_v[sl] = jnp.where(v < 0, v + nrows, v)

            e16 = lax.iota(jnp.int32, 16)

            def chunk_body(ch, _):
                cbase = pl.multiple_of(ch * _CHUNK, _CHUNK)
                v16 = ids_v[pl.ds(cbase, _CHUNK)]
                c16 = lax.shift_left(lax.shift_right_logical(v16, 7), 7)
                m16 = jnp.bitwise_and(v16, 127)
                row16 = cbase + e16
                cols = [
                    pl.multiple_of(
                        jnp.max(jnp.where(e16 == e, c16, 0)), 128)
                    for e in range(_CHUNK)
                ]
                for tr in range(ntr):
                    copies = []
                    for e in range(_CHUNK):
                        copies.append(pltpu.async_copy(
                            tbl.at[pl.ds(tr * 8, 8), pl.ds(cols[e], 128)],
                            stage.at[e], sem))
                    for cpy in copies:
                        cpy.wait()
                    for s in range(8):
                        vals = plsc.load_gather(
                            stage,
                            [e16, jnp.full((16,), s, jnp.int32), m16])
                        plsc.store_scatter(
                            rows,
                            [row16, jnp.full((16,), tr * 8 + s, jnp.int32)],
                            vals)
                return _

            lax.fori_loop(0, nch, chunk_body, 0, unroll=False)
            pltpu.sync_copy(rows, out.at[pl.ds(base, bpw)])

        table(uid_hbm, uembT_hbm, ue_out, U)
        table(iid_hbm, iembT_hbm, ie_out, I)

    return sc_gather


def _mlp_body(ue_ref, ie_ref, tp_ref, g_ref, o_ref,
              gemb_ref, oemb_ref,
              w1u_ref, w1g_ref, w1o_ref, w1i_ref, w1t_ref, b1_ref,
              w2_ref, b2_ref, w3_ref, b3_ref, out_ref):
    f32 = jnp.float32
    dot = functools.partial(jnp.dot, preferred_element_type=f32)

    acc = dot(ue_ref[...], w1u_ref[...])
    acc += dot(ie_ref[...], w1i_ref[...])
    acc += dot(tp_ref[...], w1t_ref[...])

    g2 = dot(gemb_ref[...], w1g_ref[...])
    acc += jnp.where(g_ref[...] == 0, g2[0:1, :], g2[1:2, :])

    o2 = dot(oemb_ref[...], w1o_ref[...])
    blk = o_ref.shape[0]
    iota = lax.broadcasted_iota(jnp.int32, (blk, o2.shape[0]), 1)
    oh = (o_ref[...] == iota).astype(f32)
    acc += dot(oh, o2)

    h1 = jnp.maximum(acc + b1_ref[...], 0.0)
    h2 = jnp.maximum(dot(h1, w2_ref[...]) + b2_ref[...], 0.0)
    z = dot(h2, w3_ref[...]) + b3_ref[...]
    out_ref[...] = 1.0 / (1.0 + jnp.exp(-z))


def _mlp_call(B, blk, ue, ie, tp, g2d, o2d, gemb, oemb,
              w1u, w1g, w1o, w1i, w1t, b1, w2, b2, w3, b3):
    grid = (B // blk,)

    def row_spec(c):
        return pl.BlockSpec((blk, c), lambda i: (i, 0))

    def full_spec(shape):
        return pl.BlockSpec(shape, lambda i: (0,) * len(shape))

    return pl.pallas_call(
        _mlp_body,
        grid=grid,
        in_specs=[
            row_spec(ue.shape[1]), row_spec(ie.shape[1]), row_spec(tp.shape[1]),
            row_spec(1), row_spec(1),
            full_spec(gemb.shape), full_spec(oemb.shape),
            full_spec(w1u.shape), full_spec(w1g.shape), full_spec(w1o.shape),
            full_spec(w1i.shape), full_spec(w1t.shape), full_spec(b1.shape),
            full_spec(w2.shape), full_spec(b2.shape),
            full_spec(w3.shape), full_spec(b3.shape),
        ],
        out_specs=pl.BlockSpec((blk, 1), lambda i: (i, 0)),
        out_shape=jax.ShapeDtypeStruct((B, 1), jnp.float32),
    )(ue, ie, tp, g2d, o2d, gemb, oemb,
      w1u, w1g, w1o, w1i, w1t, b1, w2, b2, w3, b3)


def kernel(x, gender, occupation, type, user_emb, item_emb, gender_emb, occ_emb,
           W1, b1, W2, b2, W3, b3):
    B = x.shape[0]
    U, H = user_emb.shape
    I = item_emb.shape[0]

    xi32 = x.astype(jnp.int32)
    ue, ie = _make_sc_gather(B, H, U, I)(
        xi32[:, 0], xi32[:, 1], user_emb.T, item_emb.T)

    Hg = gender_emb.shape[1]
    Ho = occ_emb.shape[1]
    o0 = H + Hg
    i0 = o0 + Ho
    t0 = i0 + H
    return _mlp_call(
        B, 2048, ue, ie, type,
        gender.astype(jnp.int32).reshape(B, 1),
        occupation.astype(jnp.int32).reshape(B, 1),
        gender_emb, occ_emb,
        W1[:H], W1[H:o0], W1[o0:i0], W1[i0:t0], W1[t0:], b1.reshape(1, H),
        W2, b2.reshape(1, -1), W3, b3.reshape(1, 1))


# speedup vs baseline: 6.7239x; 1.3209x over previous
import functools

import jax
import jax.numpy as jnp
from jax import lax
from jax.experimental import pallas as pl
from jax.experimental.pallas import tpu as pltpu
from jax.experimental.pallas import tpu_sc as plsc

_LANES = 16
_CHUNK = 16


@functools.lru_cache(maxsize=None)
def _make_sc_gather(B, H, U, I):
    info = plsc.get_sparse_core_info()
    NC, NS = info.num_cores, info.num_subcores
    NW = NC * NS
    bpw = B // NW
    nch = bpw // _CHUNK
    ntr = H // 8
    assert bpw * NW == B and nch * _CHUNK == bpw

    mesh = plsc.VectorSubcoreMesh(core_axis_name="c", subcore_axis_name="s")

    @functools.partial(
        pl.kernel,
        mesh=mesh,
        compiler_params=pltpu.CompilerParams(needs_layout_passes=False),
        out_type=(
            jax.ShapeDtypeStruct((B, H), jnp.float32),
            jax.ShapeDtypeStruct((B, H), jnp.float32),
        ),
        scratch_types=[
            pltpu.VMEM((bpw,), jnp.int32),
            pltpu.VMEM((_CHUNK, 8, 128), jnp.float32),
            pltpu.VMEM((_CHUNK, 8, 128), jnp.float32),
            pltpu.VMEM((bpw, H), jnp.float32),
            pltpu.SemaphoreType.DMA,
            pltpu.SemaphoreType.DMA,
        ],
    )
    def sc_gather(uid_hbm, iid_hbm, uembT_hbm, iembT_hbm, ue_out, ie_out,
                  ids_v, stage_a, stage_b, rows, sem, sem2):
        wid = lax.axis_index("s") * NC + lax.axis_index("c")
        base = pl.multiple_of(wid * bpw, bpw)

        def table(idx_hbm, tbl, out, nrows):
            pltpu.sync_copy(idx_hbm.at[pl.ds(base, bpw)], ids_v)
            for j in range(bpw // _LANES):
                sl = pl.ds(j * _LANES, _LANES)
                v = ids_v[sl] - 1
                ids_v[sl] = jnp.where(v < 0, v + nrows, v)

            e16 = lax.iota(jnp.int32, 16)

            def chunk_body(ch, _):
                cbase = pl.multiple_of(ch * _CHUNK, _CHUNK)
                v16 = ids_v[pl.ds(cbase, _CHUNK)]
                c16 = lax.shift_left(lax.shift_right_logical(v16, 7), 7)
                m16 = jnp.bitwise_and(v16, 127)
                row16 = cbase + e16
                cols = [
                    pl.multiple_of(
                        jnp.max(jnp.where(e16 == e, c16, 0)), 128)
                    for e in range(_CHUNK)
                ]
                bufs = [stage_a, stage_b]
                sems = [sem, sem2]

                def issue(tr):
                    buf, sm = bufs[tr % 2], sems[tr % 2]
                    return [
                        pltpu.async_copy(
                            tbl.at[pl.ds(tr * 8, 8), pl.ds(cols[e], 128)],
                            buf.at[e], sm)
                        for e in range(_CHUNK)
                    ]

                def extract(tr):
                    buf = bufs[tr % 2]
                    for s in range(8):
                        vals = plsc.load_gather(
                            buf,
                            [e16, jnp.full((16,), s, jnp.int32), m16])
                        plsc.store_scatter(
                            rows,
                            [row16, jnp.full((16,), tr * 8 + s, jnp.int32)],
                            vals)

                pend = {0: issue(0), 1: issue(1)}
                for tr in range(ntr):
                    for cpy in pend[tr]:
                        cpy.wait()
                    extract(tr)
                    if tr + 2 < ntr:
                        pend[tr + 2] = issue(tr + 2)
                return _

            lax.fori_loop(0, nch, chunk_body, 0, unroll=False)
            pltpu.sync_copy(rows, out.at[pl.ds(base, bpw)])

        table(uid_hbm, uembT_hbm, ue_out, U)
        table(iid_hbm, iembT_hbm, ie_out, I)

    return sc_gather


def _mlp_body(ue_ref, ie_ref, tp_ref, g_ref, o_ref,
              gemb_ref, oemb_ref,
              w1u_ref, w1g_ref, w1o_ref, w1i_ref, w1t_ref, b1_ref,
              w2_ref, b2_ref, w3_ref, b3_ref, out_ref):
    f32 = jnp.float32
    dot = functools.partial(jnp.dot, preferred_element_type=f32)

    acc = dot(ue_ref[...], w1u_ref[...])
    acc += dot(ie_ref[...], w1i_ref[...])
    acc += dot(tp_ref[...], w1t_ref[...])

    g2 = dot(gemb_ref[...], w1g_ref[...])
    acc += jnp.where(g_ref[...] == 0, g2[0:1, :], g2[1:2, :])

    o2 = dot(oemb_ref[...], w1o_ref[...])
    blk = o_ref.shape[0]
    iota = lax.broadcasted_iota(jnp.int32, (blk, o2.shape[0]), 1)
    oh = (o_ref[...] == iota).astype(f32)
    acc += dot(oh, o2)

    h1 = jnp.maximum(acc + b1_ref[...], 0.0)
    h2 = jnp.maximum(dot(h1, w2_ref[...]) + b2_ref[...], 0.0)
    z = dot(h2, w3_ref[...]) + b3_ref[...]
    out_ref[...] = 1.0 / (1.0 + jnp.exp(-z))


def _mlp_call(B, blk, ue, ie, tp, g2d, o2d, gemb, oemb,
              w1u, w1g, w1o, w1i, w1t, b1, w2, b2, w3, b3):
    grid = (B // blk,)

    def row_spec(c):
        return pl.BlockSpec((blk, c), lambda i: (i, 0))

    def full_spec(shape):
        return pl.BlockSpec(shape, lambda i: (0,) * len(shape))

    return pl.pallas_call(
        _mlp_body,
        grid=grid,
        in_specs=[
            row_spec(ue.shape[1]), row_spec(ie.shape[1]), row_spec(tp.shape[1]),
            row_spec(1), row_spec(1),
            full_spec(gemb.shape), full_spec(oemb.shape),
            full_spec(w1u.shape), full_spec(w1g.shape), full_spec(w1o.shape),
            full_spec(w1i.shape), full_spec(w1t.shape), full_spec(b1.shape),
            full_spec(w2.shape), full_spec(b2.shape),
            full_spec(w3.shape), full_spec(b3.shape),
        ],
        out_specs=pl.BlockSpec((blk, 1), lambda i: (i, 0)),
        out_shape=jax.ShapeDtypeStruct((B, 1), jnp.float32),
    )(ue, ie, tp, g2d, o2d, gemb, oemb,
      w1u, w1g, w1o, w1i, w1t, b1, w2, b2, w3, b3)


def kernel(x, gender, occupation, type, user_emb, item_emb, gender_emb, occ_emb,
           W1, b1, W2, b2, W3, b3):
    B = x.shape[0]
    U, H = user_emb.shape
    I = item_emb.shape[0]

    xi32 = x.astype(jnp.int32)
    ue, ie = _make_sc_gather(B, H, U, I)(
        xi32[:, 0], xi32[:, 1], user_emb.T, item_emb.T)

    Hg = gender_emb.shape[1]
    Ho = occ_emb.shape[1]
    o0 = H + Hg
    i0 = o0 + Ho
    t0 = i0 + H
    return _mlp_call(
        B, 2048, ue, ie, type,
        gender.astype(jnp.int32).reshape(B, 1),
        occupation.astype(jnp.int32).reshape(B, 1),
        gender_emb, occ_emb,
        W1[:H], W1[H:o0], W1[o0:i0], W1[i0:t0], W1[t0:], b1.reshape(1, H),
        W2, b2.reshape(1, -1), W3, b3.reshape(1, 1))


# speedup vs baseline: 8.1713x; 1.2153x over previous
import functools

import jax
import jax.numpy as jnp
from jax import lax
from jax.experimental import pallas as pl
from jax.experimental.pallas import tpu as pltpu
from jax.experimental.pallas import tpu_sc as plsc

_LANES = 16
_CHUNK = 16


@functools.lru_cache(maxsize=None)
def _make_sc_gather(B, H, U, I):
    info = plsc.get_sparse_core_info()
    NC, NS = info.num_cores, info.num_subcores
    NW = NC * NS
    bpw = B // NW
    nch = bpw // _CHUNK
    ntr = H // 8
    assert bpw * NW == B and nch * _CHUNK == bpw

    mesh = plsc.VectorSubcoreMesh(core_axis_name="c", subcore_axis_name="s")

    @functools.partial(
        pl.kernel,
        mesh=mesh,
        compiler_params=pltpu.CompilerParams(needs_layout_passes=False),
        out_type=(
            jax.ShapeDtypeStruct((B, H), jnp.float32),
            jax.ShapeDtypeStruct((B, H), jnp.float32),
        ),
        scratch_types=[
            pltpu.VMEM((bpw,), jnp.int32),
            pltpu.VMEM((_CHUNK, 8, 128), jnp.float32),
            pltpu.VMEM((_CHUNK, 8, 128), jnp.float32),
            pltpu.VMEM((_CHUNK, 8, 128), jnp.float32),
            pltpu.VMEM((_CHUNK, 8, 128), jnp.float32),
            pltpu.VMEM((bpw // 2, H), jnp.float32),
            pltpu.SemaphoreType.DMA,
            pltpu.SemaphoreType.DMA,
            pltpu.SemaphoreType.DMA,
            pltpu.SemaphoreType.DMA,
        ],
    )
    def sc_gather(uid_hbm, iid_hbm, uembT_hbm, iembT_hbm, ue_out, ie_out,
                  ids_v, st0, st1, st2, st3, rows, sm0, sm1, sm2, sm3):
        wid = lax.axis_index("s") * NC + lax.axis_index("c")
        base = pl.multiple_of(wid * bpw, bpw)

        bufs = [st0, st1, st2, st3]
        sems = [sm0, sm1, sm2, sm3]
        e16 = lax.iota(jnp.int32, 16)

        def table(idx_hbm, tbl, out, nrows):
            pltpu.sync_copy(idx_hbm.at[pl.ds(base, bpw)], ids_v)
            for j in range(bpw // _LANES):
                sl = pl.ds(j * _LANES, _LANES)
                v = ids_v[sl] - 1
                ids_v[sl] = jnp.where(v < 0, v + nrows, v)

            def cols_for(cbase):
                v16 = ids_v[pl.ds(cbase, _CHUNK)]
                c16 = lax.shift_left(lax.shift_right_logical(v16, 7), 7)
                return [
                    pl.multiple_of(
                        jnp.max(jnp.where(e16 == e, c16, 0)), 128)
                    for e in range(_CHUNK)
                ]

            def issue(tr, cols):
                for e in range(_CHUNK):
                    pltpu.async_copy(
                        tbl.at[pl.ds(tr * 8, 8), pl.ds(cols[e], 128)],
                        bufs[tr].at[e], sems[tr])

            def drain(tr):
                for e in range(_CHUNK):
                    pltpu.make_async_copy(
                        tbl.at[pl.ds(0, 8), pl.ds(0, 128)],
                        bufs[tr].at[e], sems[tr]).wait()

            cols0 = cols_for(0)
            for tr in range(ntr):
                issue(tr, cols0)

            def make_chunk_body(half):
                def chunk_body(ch, _):
                    cbase = pl.multiple_of(ch * _CHUNK, _CHUNK)
                    v16 = ids_v[pl.ds(cbase, _CHUNK)]
                    m16 = jnp.bitwise_and(v16, 127)
                    row16 = cbase - half * (bpw // 2) + e16
                    nbase = pl.multiple_of(
                        jnp.minimum(ch + 1, nch - 1) * _CHUNK, _CHUNK)
                    colsN = cols_for(nbase)
                    for tr in range(ntr):
                        drain(tr)
                        for s in range(8):
                            vals = plsc.load_gather(
                                bufs[tr],
                                [e16, jnp.full((16,), s, jnp.int32), m16])
                            plsc.store_scatter(
                                rows,
                                [row16,
                                 jnp.full((16,), tr * 8 + s, jnp.int32)],
                                vals)
                        issue(tr, colsN)
                    return _
                return chunk_body

            for half in range(2):
                lax.fori_loop(half * nch // 2, (half + 1) * nch // 2,
                              make_chunk_body(half), 0, unroll=False)
                pltpu.sync_copy(
                    rows, out.at[pl.ds(base + half * (bpw // 2), bpw // 2)])
            for tr in range(ntr):
                drain(tr)

        table(uid_hbm, uembT_hbm, ue_out, U)
        table(iid_hbm, iembT_hbm, ie_out, I)

    return sc_gather


def _mlp_body(ue_ref, ie_ref, tp_ref, g_ref, o_ref,
              gemb_ref, oemb_ref,
              w1u_ref, w1g_ref, w1o_ref, w1i_ref, w1t_ref, b1_ref,
              w2_ref, b2_ref, w3_ref, b3_ref, out_ref):
    f32 = jnp.float32
    dot = functools.partial(jnp.dot, preferred_element_type=f32)

    acc = dot(ue_ref[...], w1u_ref[...])
    acc += dot(ie_ref[...], w1i_ref[...])
    acc += dot(tp_ref[...], w1t_ref[...])

    g2 = dot(gemb_ref[...], w1g_ref[...])
    acc += jnp.where(g_ref[...] == 0, g2[0:1, :], g2[1:2, :])

    o2 = dot(oemb_ref[...], w1o_ref[...])
    blk = o_ref.shape[0]
    iota = lax.broadcasted_iota(jnp.int32, (blk, o2.shape[0]), 1)
    oh = (o_ref[...] == iota).astype(f32)
    acc += dot(oh, o2)

    h1 = jnp.maximum(acc + b1_ref[...], 0.0)
    h2 = jnp.maximum(dot(h1, w2_ref[...]) + b2_ref[...], 0.0)
    z = dot(h2, w3_ref[...]) + b3_ref[...]
    out_ref[...] = 1.0 / (1.0 + jnp.exp(-z))


def _mlp_call(B, blk, ue, ie, tp, g2d, o2d, gemb, oemb,
              w1u, w1g, w1o, w1i, w1t, b1, w2, b2, w3, b3):
    grid = (B // blk,)

    def row_spec(c):
        return pl.BlockSpec((blk, c), lambda i: (i, 0))

    def full_spec(shape):
        return pl.BlockSpec(shape, lambda i: (0,) * len(shape))

    return pl.pallas_call(
        _mlp_body,
        grid=grid,
        in_specs=[
            row_spec(ue.shape[1]), row_spec(ie.shape[1]), row_spec(tp.shape[1]),
            row_spec(1), row_spec(1),
            full_spec(gemb.shape), full_spec(oemb.shape),
            full_spec(w1u.shape), full_spec(w1g.shape), full_spec(w1o.shape),
            full_spec(w1i.shape), full_spec(w1t.shape), full_spec(b1.shape),
            full_spec(w2.shape), full_spec(b2.shape),
            full_spec(w3.shape), full_spec(b3.shape),
        ],
        out_specs=pl.BlockSpec((blk, 1), lambda i: (i, 0)),
        out_shape=jax.ShapeDtypeStruct((B, 1), jnp.float32),
    )(ue, ie, tp, g2d, o2d, gemb, oemb,
      w1u, w1g, w1o, w1i, w1t, b1, w2, b2, w3, b3)


def kernel(x, gender, occupation, type, user_emb, item_emb, gender_emb, occ_emb,
           W1, b1, W2, b2, W3, b3):
    B = x.shape[0]
    U, H = user_emb.shape
    I = item_emb.shape[0]

    xi32 = x.astype(jnp.int32)
    ue, ie = _make_sc_gather(B, H, U, I)(
        xi32[:, 0], xi32[:, 1], user_emb.T, item_emb.T)

    Hg = gender_emb.shape[1]
    Ho = occ_emb.shape[1]
    o0 = H + Hg
    i0 = o0 + Ho
    t0 = i0 + H
    return _mlp_call(
        B, 2048, ue, ie, type,
        gender.astype(jnp.int32).reshape(B, 1),
        occupation.astype(jnp.int32).reshape(B, 1),
        gender_emb, occ_emb,
        W1[:H], W1[H:o0], W1[o0:i0], W1[i0:t0], W1[t0:], b1.reshape(1, H),
        W2, b2.reshape(1, -1), W3, b3.reshape(1, 1))
